# baseline (speedup 1.0000x reference)
import functools

import jax
import jax.numpy as jnp
from jax import lax
from jax.experimental import pallas as pl
from jax.experimental.pallas import tpu as pltpu
from jax.experimental.pallas import tpu_sc as plsc

DIM = 64
NI = 16384
NJ = 50
NC = 2
NS = 16
NW = NC * NS
CH = 128
NIB = NI // CH
NBLK = NJ * NIB
BPW = NBLK // NW
NBUF = 4
L = 16

NROW = 1000000
NMAIN = 999936
CCOL = 384
CPAIR = CCOL // 2
NCHK = NMAIN // CCOL
NAPP = (NROW - NMAIN) // 2

_mesh = plsc.VectorSubcoreMesh(core_axis_name="c", subcore_axis_name="s")


@functools.partial(
    pl.kernel,
    mesh=_mesh,
    out_type=jax.ShapeDtypeStruct((NROW // 2, 128), jnp.float32),
    scratch_types=[
        pltpu.VMEM((2, DIM, CCOL), jnp.float32),
        pltpu.VMEM((2, CPAIR, 128), jnp.float32),
    ]
    + [pltpu.SemaphoreType.DMA] * 4,
    compiler_params=pltpu.CompilerParams(use_tc_tiling_on_sc=True,
                                         needs_layout_passes=False),
)
def _conv(wt_hbm, app_hbm, out_hbm, bufa, bufb, *sems):
    gsem = sems[:2]
    osem = sems[2:]
    wid = lax.axis_index("s") * NC + lax.axis_index("c")

    @pl.when(wid == NW - 1)
    def _():
        pltpu.sync_copy(app_hbm, bufb.at[0, pl.ds(0, NAPP)])
        pltpu.sync_copy(bufb.at[0, pl.ds(0, NAPP)],
                        out_hbm.at[pl.ds(NMAIN // 2, NAPP), :])

    def in_copy(g, s):
        return pltpu.make_async_copy(
            wt_hbm.at[:, pl.ds(g * CCOL, CCOL)], bufa.at[s], gsem[s])

    def out_copy(g, s):
        return pltpu.make_async_copy(
            bufb.at[s], out_hbm.at[pl.ds(g * CPAIR, CPAIR), :], osem[s])

    def fire(t, s):
        g = wid + NW * t

        @pl.when(g < NCHK)
        def _():
            in_copy(g, s).start()

    r_vecs = [lax.iota(jnp.int32, L) + u * L for u in range(4)]

    def transpose(s):
        @plsc.parallel_loop(0, CPAIR, unroll=2)
        def _(p):
            for u in range(8):
                cvec = jnp.zeros((L,), jnp.int32) + (2 * p + (u // 4))
                vals = plsc.load_gather(bufa.at[s], [r_vecs[u % 4], cvec])
                bufb[s, p, pl.ds(u * L, L)] = vals

    fire(0, 0)
    fire(1, 1)

    def body(tt, carry):
        for s in range(2):
            t = 2 * tt + s
            g = wid + NW * t

            @pl.when(g < NCHK)
            def _():
                in_copy(g, s).wait()

                @pl.when(t >= 2)
                def _():
                    out_copy(g, s).wait()
                transpose(s)
                out_copy(g, s).start()
                fire(t + 2, s)
        return carry

    lax.fori_loop(0, 41, body, 0)
    out_copy(0, 0).wait()
    out_copy(0, 1).wait()


@functools.partial(
    pl.kernel,
    mesh=_mesh,
    out_type=jax.ShapeDtypeStruct((NJ, DIM, NI), jnp.float32),
    scratch_types=[
        pltpu.VMEM((BPW, CH), jnp.int32),
        pltpu.VMEM((NBUF, CH), jnp.int32),
        pltpu.VMEM((NBUF, CH, 128), jnp.float32),
        pltpu.VMEM((NBUF, DIM, CH), jnp.float32),
    ]
    + [pltpu.SemaphoreType.DMA] * (2 * NBUF),
    compiler_params=pltpu.CompilerParams(use_tc_tiling_on_sc=True,
                                         needs_layout_passes=False),
)
def _embed(w2_hbm, xt_hbm, out_hbm, idx_v, gidx_v, rows_v, ostage_v,
           *sems):
    gsems = sems[:NBUF]
    osems = sems[NBUF:]
    wid = lax.axis_index("s") * NC + lax.axis_index("c")
    blk0 = wid * BPW

    pltpu.sync_copy(xt_hbm.at[pl.ds(blk0, BPW)], idx_v)

    def fire(k, b):
        for g in range(CH // L):
            raw = idx_v[k, pl.ds(g * L, L)]
            gidx_v[b, pl.ds(g * L, L)] = lax.shift_right_logical(raw, 1)
        pltpu.make_async_copy(
            w2_hbm.at[gidx_v.at[b]], rows_v.at[b], gsems[b]).start()

    def out_block(k):
        j = (blk0 + k) // NIB
        ib = (blk0 + k) % NIB
        return out_hbm.at[j, :, pl.ds(ib * CH, CH)]

    l_vecs = [lax.iota(jnp.int32, L) + g * L for g in range(CH // L)]

    def select(k, b):
        cols = [(idx_v[k, pl.ds(g * L, L)] & 1) * 64 for g in range(CH // L)]

        @plsc.parallel_loop(0, DIM, unroll=4)
        def _(c):
            for g in range(CH // L):
                vals = plsc.load_gather(rows_v.at[b], [l_vecs[g], cols[g] + c])
                ostage_v[b, c, pl.ds(g * L, L)] = vals

    for b in range(NBUF):
        fire(b, b)

    def outer(g, carry):
        for b in range(NBUF):
            k = g * NBUF + b
            pltpu.make_async_copy(
                w2_hbm.at[gidx_v.at[b]], rows_v.at[b], gsems[b]).wait()

            @pl.when(k >= NBUF)
            def _():
                pltpu.make_async_copy(
                    ostage_v.at[b], out_block(k - NBUF), osems[b]).wait()
            select(k, b)
            pltpu.make_async_copy(
                ostage_v.at[b], out_block(k), osems[b]).start()

            @pl.when(k + NBUF < BPW)
            def _():
                fire(k + NBUF, b)
        return carry

    lax.fori_loop(0, BPW // NBUF, outer, 0)
    for b in range(NBUF):
        k = BPW - NBUF + b
        pltpu.make_async_copy(
            ostage_v.at[b], out_block(k), osems[b]).wait()


def kernel(x, W):
    wt = jnp.transpose(W)
    app = jnp.reshape(W[NMAIN:], (NAPP, 128))
    w2 = _conv(wt, app)
    xt = jnp.reshape(jnp.transpose(x.astype(jnp.int32)), (NBLK, CH))
    out = _embed(w2, xt)
    return jnp.transpose(out, (2, 0, 1))


# speedup vs baseline: 1.3729x; 1.3729x over previous
import functools

import jax
import jax.numpy as jnp
from jax import lax
from jax.experimental import pallas as pl
from jax.experimental.pallas import tpu as pltpu
from jax.experimental.pallas import tpu_sc as plsc

DIM = 64
NI = 16384
NJ = 50
NC = 2
NS = 16
NW = NC * NS
CH = 128
NIB = NI // CH
NBLK = NJ * NIB
BPW = NBLK // NW
NBUF = 4
L = 16

NROW = 1000000
NMAIN = 999936
CCOL = 384
CPAIR = CCOL // 2
NCHK = NMAIN // CCOL
NAPP = (NROW - NMAIN) // 2

_mesh = plsc.VectorSubcoreMesh(core_axis_name="c", subcore_axis_name="s")


@functools.partial(
    pl.kernel,
    mesh=_mesh,
    out_type=jax.ShapeDtypeStruct((NROW // 2, 128), jnp.float32),
    scratch_types=[
        pltpu.VMEM((2, DIM, CCOL), jnp.float32),
        pltpu.VMEM((2, CPAIR, 128), jnp.float32),
    ]
    + [pltpu.SemaphoreType.DMA] * 4,
    compiler_params=pltpu.CompilerParams(use_tc_tiling_on_sc=True,
                                         needs_layout_passes=False),
)
def _conv(wt_hbm, app_hbm, out_hbm, bufa, bufb, *sems):
    gsem = sems[:2]
    osem = sems[2:]
    wid = lax.axis_index("s") * NC + lax.axis_index("c")

    @pl.when(wid == NW - 1)
    def _():
        pltpu.sync_copy(app_hbm, bufb.at[0, pl.ds(0, NAPP)])
        pltpu.sync_copy(bufb.at[0, pl.ds(0, NAPP)],
                        out_hbm.at[pl.ds(NMAIN // 2, NAPP), :])

    def in_copy(g, s):
        return pltpu.make_async_copy(
            wt_hbm.at[:, pl.ds(g * CCOL, CCOL)], bufa.at[s], gsem[s])

    def out_copy(g, s):
        return pltpu.make_async_copy(
            bufb.at[s], out_hbm.at[pl.ds(g * CPAIR, CPAIR), :], osem[s])

    def fire(t, s):
        g = wid + NW * t

        @pl.when(g < NCHK)
        def _():
            in_copy(g, s).start()

    r_vecs = [lax.iota(jnp.int32, L) + u * L for u in range(4)]

    def transpose(s):
        @plsc.parallel_loop(0, CPAIR, unroll=2)
        def _(p):
            for u in range(8):
                cvec = jnp.zeros((L,), jnp.int32) + (2 * p + (u // 4))
                vals = plsc.load_gather(bufa.at[s], [r_vecs[u % 4], cvec])
                bufb[s, p, pl.ds(u * L, L)] = vals

    fire(0, 0)
    fire(1, 1)

    def body(tt, carry):
        for s in range(2):
            t = 2 * tt + s
            g = wid + NW * t

            @pl.when(g < NCHK)
            def _():
                in_copy(g, s).wait()

                @pl.when(t >= 2)
                def _():
                    out_copy(g, s).wait()
                transpose(s)
                out_copy(g, s).start()
                fire(t + 2, s)
        return carry

    lax.fori_loop(0, 41, body, 0)
    out_copy(0, 0).wait()
    out_copy(0, 1).wait()


@functools.partial(
    pl.kernel,
    mesh=_mesh,
    out_type=jax.ShapeDtypeStruct((NJ, DIM, NI), jnp.float32),
    scratch_types=[
        pltpu.VMEM((BPW, CH), jnp.int32),
        pltpu.VMEM((NBUF, CH), jnp.int32),
        pltpu.VMEM((NBUF, CH, 128), jnp.float32),
        pltpu.VMEM((NBUF, DIM, CH), jnp.float32),
    ]
    + [pltpu.SemaphoreType.DMA] * (2 * NBUF),
    compiler_params=pltpu.CompilerParams(use_tc_tiling_on_sc=True,
                                         needs_layout_passes=False),
)
def _embed(w2_hbm, xt_hbm, out_hbm, idx_v, gidx_v, rows_v, ostage_v,
           *sems):
    gsems = sems[:NBUF]
    osems = sems[NBUF:]
    wid = lax.axis_index("s") * NC + lax.axis_index("c")
    blk0 = wid * BPW

    pltpu.sync_copy(xt_hbm.at[pl.ds(blk0, BPW)], idx_v)

    def fire(k, b):
        for g in range(CH // L):
            raw = idx_v[k, pl.ds(g * L, L)]
            gidx_v[b, pl.ds(g * L, L)] = lax.shift_right_logical(raw, 1)
        pltpu.make_async_copy(
            w2_hbm.at[gidx_v.at[b]], rows_v.at[b], gsems[b]).start()

    def out_block(k):
        j = (blk0 + k) // NIB
        ib = (blk0 + k) % NIB
        return out_hbm.at[j, :, pl.ds(ib * CH, CH)]

    l_vecs = [lax.iota(jnp.int32, L) + g * L for g in range(CH // L)]

    iota = lax.iota(jnp.int32, L)

    def select(k, b):
        pars = [(idx_v[k, pl.ds(g * L, L)] & 1) * 64 for g in range(CH // L)]

        @plsc.parallel_loop(0, L, unroll=2)
        def _(r):
            rot = (iota + r) & (L - 1)
            for g in range(CH // L):
                base = pars[g] + rot
                for cb in range(DIM // L):
                    vals = plsc.load_gather(
                        rows_v.at[b], [l_vecs[g], base + cb * L])
                    plsc.store_scatter(
                        ostage_v.at[b], [rot + cb * L, l_vecs[g]], vals)

    for b in range(NBUF):
        fire(b, b)

    def outer(g, carry):
        for b in range(NBUF):
            k = g * NBUF + b
            pltpu.make_async_copy(
                w2_hbm.at[gidx_v.at[b]], rows_v.at[b], gsems[b]).wait()

            @pl.when(k >= NBUF)
            def _():
                pltpu.make_async_copy(
                    ostage_v.at[b], out_block(k - NBUF), osems[b]).wait()
            select(k, b)
            pltpu.make_async_copy(
                ostage_v.at[b], out_block(k), osems[b]).start()

            @pl.when(k + NBUF < BPW)
            def _():
                fire(k + NBUF, b)
        return carry

    lax.fori_loop(0, BPW // NBUF, outer, 0)
    for b in range(NBUF):
        k = BPW - NBUF + b
        pltpu.make_async_copy(
            ostage_v.at[b], out_block(k), osems[b]).wait()


def kernel(x, W):
    wt = jnp.transpose(W)
    app = jnp.reshape(W[NMAIN:], (NAPP, 128))
    w2 = _conv(wt, app)
    xt = jnp.reshape(jnp.transpose(x.astype(jnp.int32)), (NBLK, CH))
    out = _embed(w2, xt)
    return jnp.transpose(out, (2, 0, 1))


# speedup vs baseline: 2.1239x; 1.5470x over previous
import functools

import jax
import jax.numpy as jnp
from jax import lax
from jax.experimental import pallas as pl
from jax.experimental.pallas import tpu as pltpu
from jax.experimental.pallas import tpu_sc as plsc

DIM = 64
NI = 16384
NJ = 50
NC = 2
NS = 16
NW = NC * NS
CH = 128
NIB = NI // CH
NBLK = NJ * NIB
BPW = NBLK // NW
NBUF = 4
L = 16

NROW = 1000000
NMAIN = 999936
CCOL = 384
CPAIR = CCOL // 2
NCHK = NMAIN // CCOL
NAPP = (NROW - NMAIN) // 2

_mesh = plsc.VectorSubcoreMesh(core_axis_name="c", subcore_axis_name="s")


@functools.partial(
    pl.kernel,
    mesh=_mesh,
    out_type=jax.ShapeDtypeStruct((NROW // 2, 128), jnp.float32),
    scratch_types=[
        pltpu.VMEM((2, DIM, CCOL), jnp.float32),
        pltpu.VMEM((2, CPAIR, 128), jnp.float32),
    ]
    + [pltpu.SemaphoreType.DMA] * 4,
    compiler_params=pltpu.CompilerParams(use_tc_tiling_on_sc=True,
                                         needs_layout_passes=False),
)
def _conv(wt_hbm, app_hbm, out_hbm, bufa, bufb, *sems):
    gsem = sems[:2]
    osem = sems[2:]
    wid = lax.axis_index("s") * NC + lax.axis_index("c")

    @pl.when(wid == NW - 1)
    def _():
        pltpu.sync_copy(app_hbm, bufb.at[0, pl.ds(0, NAPP)])
        pltpu.sync_copy(bufb.at[0, pl.ds(0, NAPP)],
                        out_hbm.at[pl.ds(NMAIN // 2, NAPP), :])

    def in_copy(g, s):
        return pltpu.make_async_copy(
            wt_hbm.at[:, pl.ds(g * CCOL, CCOL)], bufa.at[s], gsem[s])

    def out_copy(g, s):
        return pltpu.make_async_copy(
            bufb.at[s], out_hbm.at[pl.ds(g * CPAIR, CPAIR), :], osem[s])

    def fire(t, s):
        g = wid + NW * t

        @pl.when(g < NCHK)
        def _():
            in_copy(g, s).start()

    iota = lax.iota(jnp.int32, L)

    def transpose(s):
        @plsc.parallel_loop(0, L, unroll=2)
        def _(r):
            rot = (iota + r) & (L - 1)
            for q0 in range(0, 128, L):
                row_vec = (q0 % 64) + rot
                col_q = q0 // 64
                for p0 in range(0, CPAIR, L):
                    vals = plsc.load_gather(
                        bufa.at[s], [row_vec, 2 * iota + (2 * p0 + col_q)])
                    plsc.store_scatter(
                        bufb.at[s], [p0 + iota, q0 + rot], vals)

    fire(0, 0)
    fire(1, 1)

    def body(tt, carry):
        for s in range(2):
            t = 2 * tt + s
            g = wid + NW * t

            @pl.when(g < NCHK)
            def _():
                in_copy(g, s).wait()

                @pl.when(t >= 2)
                def _():
                    out_copy(g, s).wait()
                transpose(s)
                out_copy(g, s).start()
                fire(t + 2, s)
        return carry

    lax.fori_loop(0, 41, body, 0)
    out_copy(0, 0).wait()
    out_copy(0, 1).wait()


@functools.partial(
    pl.kernel,
    mesh=_mesh,
    out_type=jax.ShapeDtypeStruct((NJ, DIM, NI), jnp.float32),
    scratch_types=[
        pltpu.VMEM((BPW, CH), jnp.int32),
        pltpu.VMEM((NBUF, CH), jnp.int32),
        pltpu.VMEM((NBUF, CH, 128), jnp.float32),
        pltpu.VMEM((NBUF, DIM, CH), jnp.float32),
    ]
    + [pltpu.SemaphoreType.DMA] * (2 * NBUF),
    compiler_params=pltpu.CompilerParams(use_tc_tiling_on_sc=True,
                                         needs_layout_passes=False),
)
def _embed(w2_hbm, xt_hbm, out_hbm, idx_v, gidx_v, rows_v, ostage_v,
           *sems):
    gsems = sems[:NBUF]
    osems = sems[NBUF:]
    wid = lax.axis_index("s") * NC + lax.axis_index("c")
    blk0 = wid * BPW

    pltpu.sync_copy(xt_hbm.at[pl.ds(blk0, BPW)], idx_v)

    def fire(k, b):
        for g in range(CH // L):
            raw = idx_v[k, pl.ds(g * L, L)]
            gidx_v[b, pl.ds(g * L, L)] = lax.shift_right_logical(raw, 1)
        pltpu.make_async_copy(
            w2_hbm.at[gidx_v.at[b]], rows_v.at[b], gsems[b]).start()

    def out_block(k):
        j = (blk0 + k) // NIB
        ib = (blk0 + k) % NIB
        return out_hbm.at[j, :, pl.ds(ib * CH, CH)]

    l_vecs = [lax.iota(jnp.int32, L) + g * L for g in range(CH // L)]

    iota = lax.iota(jnp.int32, L)

    def select(k, b):
        pars = [(idx_v[k, pl.ds(g * L, L)] & 1) * 64 for g in range(CH // L)]

        @plsc.parallel_loop(0, L, unroll=2)
        def _(r):
            rot = (iota + r) & (L - 1)
            for g in range(CH // L):
                base = pars[g] + rot
                for cb in range(DIM // L):
                    vals = plsc.load_gather(
                        rows_v.at[b], [l_vecs[g], base + cb * L])
                    plsc.store_scatter(
                        ostage_v.at[b], [rot + cb * L, l_vecs[g]], vals)

    for b in range(NBUF):
        fire(b, b)

    def outer(g, carry):
        for b in range(NBUF):
            k = g * NBUF + b
            pltpu.make_async_copy(
                w2_hbm.at[gidx_v.at[b]], rows_v.at[b], gsems[b]).wait()

            @pl.when(k >= NBUF)
            def _():
                pltpu.make_async_copy(
                    ostage_v.at[b], out_block(k - NBUF), osems[b]).wait()
            select(k, b)
            pltpu.make_async_copy(
                ostage_v.at[b], out_block(k), osems[b]).start()

            @pl.when(k + NBUF < BPW)
            def _():
                fire(k + NBUF, b)
        return carry

    lax.fori_loop(0, BPW // NBUF, outer, 0)
    for b in range(NBUF):
        k = BPW - NBUF + b
        pltpu.make_async_copy(
            ostage_v.at[b], out_block(k), osems[b]).wait()


def kernel(x, W):
    wt = jnp.transpose(W)
    app = jnp.reshape(W[NMAIN:], (NAPP, 128))
    w2 = _conv(wt, app)
    xt = jnp.reshape(jnp.transpose(x.astype(jnp.int32)), (NBLK, CH))
    out = _embed(w2, xt)
    return jnp.transpose(out, (2, 0, 1))
